# baseline (speedup 1.0000x reference)
import functools

import jax
import jax.numpy as jnp
from jax.experimental import pallas as pl
import jax.experimental.pallas.tpu as pltpu

L = 384
D = 640
EPS = 1e-5

TI = 96
TJ = 128


def _ln(x, g, b):
    m = jnp.mean(x, axis=-1, keepdims=True)
    c = x - m
    v = jnp.mean(c * c, axis=-1, keepdims=True)
    return c * jax.lax.rsqrt(v + EPS) * g + b


def _rows_kernel(x_ref, sp_w_ref, sp_b_ref, sp_g_ref, sp_be_ref,
                 dms_w1_ref, dms_b1_ref, dms_w2r_ref, dms_b2_ref,
                 sh_w1_ref, sh_b1_ref, sh_w2r_ref, sh_b2_ref,
                 ts_w1_ref, ts_b1_ref, ts_g_ref, ts_be_ref,
                 ts_w2_ref, ts_b2_ref,
                 wa_ref, wb_ref, wc_ref, tp_b1_ref,
                 tp_gc_ref, tp_be_ref, tp_w2_ref, tp_b2_ref,
                 single_ref, d_ref, s_ref, ai_ref, bj_ref, d1_ref, s1_ref,
                 dsc_ref, w2a_ref, u_ref, cw2_ref):
    x = x_ref[...]
    f32 = jnp.float32

    shared = _ln(jax.nn.relu(jnp.dot(x, sp_w_ref[...], preferred_element_type=f32)
                             + sp_b_ref[...]),
                 sp_g_ref[...], sp_be_ref[...])

    dh = jax.nn.relu(jnp.dot(shared, dms_w1_ref[...], preferred_element_type=f32)
                     + dms_b1_ref[...])
    dcol = jnp.sum(dh * dms_w2r_ref[...], axis=-1, keepdims=True) + dms_b2_ref[...]

    sh = jax.nn.relu(jnp.dot(shared, sh_w1_ref[...], preferred_element_type=f32)
                     + sh_b1_ref[...])
    scol = jnp.sum(sh * sh_w2r_ref[...], axis=-1, keepdims=True) + sh_b2_ref[...]

    t = _ln(jax.nn.relu(jnp.dot(x, ts_w1_ref[...], preferred_element_type=f32)
                        + ts_b1_ref[...]),
            ts_g_ref[...], ts_be_ref[...])
    single_ref[...] = (jnp.dot(t, ts_w2_ref[...], preferred_element_type=f32)
                       + ts_b2_ref[...]).reshape(1, L, 384)

    hi = jnp.dot(x, wa_ref[...], preferred_element_type=f32)
    hj = jnp.dot(x, wb_ref[...], preferred_element_type=f32)
    wc = wc_ref[...]
    bf16 = jnp.bfloat16
    corr = dcol * wc[0:1, :] + scol * wc[2:3, :]
    ai_ref[...] = (hi + corr).astype(bf16)
    bj_ref[...] = (hj - corr + tp_b1_ref[...]).astype(bf16)
    d1_ref[...] = (dcol * wc[1:2, :]).astype(bf16)
    s1_ref[...] = (scol * wc[3:4, :]).astype(bf16)
    d_ref[...] = dcol
    s_ref[...] = scol
    dsc_ref[...] = jnp.concatenate([jnp.broadcast_to(dcol, (L, 128)),
                                    jnp.broadcast_to(scol, (L, 128))], axis=1)

    w2 = tp_w2_ref[...]
    w2g = w2 * tp_gc_ref[...]
    w2a_ref[...] = w2g.astype(bf16)
    u_ref[...] = jnp.sum(w2g, axis=0, keepdims=True)
    cw2_ref[...] = (jnp.dot(tp_be_ref[...], w2, preferred_element_type=f32)
                    + tp_b2_ref[...])


def _pair_kernel(ai_ref, d1_ref, s1_ref, bj_ref, dsc_ref,
                 w2a_ref, u_ref, cw2_ref, w3_ref, b3_ref, o_ref, rb_ref):
    f32 = jnp.float32
    bf16 = jnp.bfloat16
    ai = ai_ref[...]
    bj = bj_ref[...]
    dj = dsc_ref[:, 0:1].astype(bf16)
    sj = dsc_ref[:, 128:129].astype(bf16)

    h = (ai[:, None, :] + bj[None, :, :]
         + d1_ref[...][:, None, :] * dj[None, :, :]
         + s1_ref[...][:, None, :] * sj[None, :, :])
    rb_ref[...] = jax.nn.relu(h).reshape(TI * TJ, 512)
    rb = rb_ref[...]
    t = jnp.dot(rb, w2a_ref[...], preferred_element_type=f32)
    r = rb.astype(f32)
    s1 = jnp.sum(r, axis=-1, keepdims=True)
    s2 = jnp.sum(r * r, axis=-1, keepdims=True)
    m = s1 * (1.0 / 512.0)
    v = s2 * (1.0 / 512.0) - m * m
    inv = jax.lax.rsqrt(v + EPS)
    h2 = jax.nn.relu(inv * (t - m * u_ref[...]) + cw2_ref[...])
    o = jnp.dot(h2, w3_ref[...], preferred_element_type=f32) + b3_ref[...]
    o_ref[...] = o.reshape(1, TI, TJ, 128)


@jax.jit
def kernel(features, sp_w, sp_b, sp_g, sp_be, dms_w1, dms_b1, dms_w2, dms_b2,
           sh_w1, sh_b1, sh_w2, sh_b2, ts_w1, ts_b1, ts_g, ts_be, ts_w2, ts_b2,
           tp_w1, tp_b1, tp_g, tp_be, tp_w2, tp_b2, tp_w3, tp_b3):
    f32 = jnp.float32
    x = features.reshape(L, D)

    row = lambda v: v.reshape(1, -1)
    wa = tp_w1[:640]
    wb = tp_w1[640:1280]
    wc = jnp.pad(tp_w1[1280:1284], ((0, 4), (0, 0)))

    vmem = lambda: pl.BlockSpec(memory_space=pltpu.VMEM)
    single, dcol, scol, ai, bj, d1, s1, dsc, w2a, u, cw2 = pl.pallas_call(
        _rows_kernel,
        out_shape=[
            jax.ShapeDtypeStruct((1, L, 384), f32),
            jax.ShapeDtypeStruct((L, 1), f32),
            jax.ShapeDtypeStruct((L, 1), f32),
            jax.ShapeDtypeStruct((L, 512), jnp.bfloat16),
            jax.ShapeDtypeStruct((L, 512), jnp.bfloat16),
            jax.ShapeDtypeStruct((L, 512), jnp.bfloat16),
            jax.ShapeDtypeStruct((L, 512), jnp.bfloat16),
            jax.ShapeDtypeStruct((L, 256), f32),
            jax.ShapeDtypeStruct((512, 256), jnp.bfloat16),
            jax.ShapeDtypeStruct((1, 256), f32),
            jax.ShapeDtypeStruct((1, 256), f32),
        ],
        in_specs=[vmem() for _ in range(27)],
        out_specs=[vmem() for _ in range(11)],
    )(x, sp_w, row(sp_b), row(sp_g), row(sp_be),
      dms_w1, row(dms_b1), dms_w2.reshape(1, 160), dms_b2.reshape(1, 1),
      sh_w1, row(sh_b1), sh_w2.reshape(1, 160), sh_b2.reshape(1, 1),
      ts_w1, row(ts_b1), row(ts_g), row(ts_be), ts_w2, row(ts_b2),
      wa, wb, wc, row(tp_b1),
      tp_g.reshape(512, 1), row(tp_be), tp_w2, row(tp_b2))

    grid = (L // TI, L // TJ)
    pair = pl.pallas_call(
        _pair_kernel,
        grid=grid,
        in_specs=[
            pl.BlockSpec((TI, 512), lambda i, j: (i, 0)),
            pl.BlockSpec((TI, 512), lambda i, j: (i, 0)),
            pl.BlockSpec((TI, 512), lambda i, j: (i, 0)),
            pl.BlockSpec((TJ, 512), lambda i, j: (j, 0)),
            pl.BlockSpec((TJ, 256), lambda i, j: (j, 0)),
            pl.BlockSpec((512, 256), lambda i, j: (0, 0)),
            pl.BlockSpec((1, 256), lambda i, j: (0, 0)),
            pl.BlockSpec((1, 256), lambda i, j: (0, 0)),
            pl.BlockSpec((256, 128), lambda i, j: (0, 0)),
            pl.BlockSpec((1, 128), lambda i, j: (0, 0)),
        ],
        out_specs=pl.BlockSpec((1, TI, TJ, 128), lambda i, j: (0, i, j, 0)),
        out_shape=jax.ShapeDtypeStruct((1, L, L, 128), f32),
        scratch_shapes=[pltpu.VMEM((TI * TJ, 512), jnp.bfloat16)],
        compiler_params=pltpu.CompilerParams(
            dimension_semantics=("parallel", "arbitrary"),
            vmem_limit_bytes=56 * 1024 * 1024,
        ),
    )(ai, d1, s1, bj, dsc, w2a, u, cw2, tp_w3, row(tp_b3))

    return (single, pair, dcol.reshape(1, L), scol.reshape(1, L))


# speedup vs baseline: 1.0123x; 1.0123x over previous
import functools

import jax
import jax.numpy as jnp
from jax.experimental import pallas as pl
import jax.experimental.pallas.tpu as pltpu

L = 384
D = 640
EPS = 1e-5

TI = 96
TJ = 128


def _ln(x, g, b):
    m = jnp.mean(x, axis=-1, keepdims=True)
    c = x - m
    v = jnp.mean(c * c, axis=-1, keepdims=True)
    return c * jax.lax.rsqrt(v + EPS) * g + b


def _rows_kernel(x_ref, sp_w_ref, sp_b_ref, sp_g_ref, sp_be_ref,
                 dms_w1_ref, dms_b1_ref, dms_w2r_ref, dms_b2_ref,
                 sh_w1_ref, sh_b1_ref, sh_w2r_ref, sh_b2_ref,
                 ts_w1_ref, ts_b1_ref, ts_g_ref, ts_be_ref,
                 ts_w2_ref, ts_b2_ref,
                 wa_ref, wb_ref, wc_ref, tp_b1_ref,
                 tp_gc_ref, tp_be_ref, tp_w2_ref, tp_b2_ref,
                 single_ref, d_ref, s_ref, ai_ref, bj_ref, d1_ref, s1_ref,
                 dsc_ref, w2a_ref, u_ref, cw2_ref):
    x = x_ref[...]
    f32 = jnp.float32

    shared = _ln(jax.nn.relu(jnp.dot(x, sp_w_ref[...], preferred_element_type=f32)
                             + sp_b_ref[...]),
                 sp_g_ref[...], sp_be_ref[...])

    dh = jax.nn.relu(jnp.dot(shared, dms_w1_ref[...], preferred_element_type=f32)
                     + dms_b1_ref[...])
    dcol = jnp.sum(dh * dms_w2r_ref[...], axis=-1, keepdims=True) + dms_b2_ref[...]

    sh = jax.nn.relu(jnp.dot(shared, sh_w1_ref[...], preferred_element_type=f32)
                     + sh_b1_ref[...])
    scol = jnp.sum(sh * sh_w2r_ref[...], axis=-1, keepdims=True) + sh_b2_ref[...]

    t = _ln(jax.nn.relu(jnp.dot(x, ts_w1_ref[...], preferred_element_type=f32)
                        + ts_b1_ref[...]),
            ts_g_ref[...], ts_be_ref[...])
    single_ref[...] = (jnp.dot(t, ts_w2_ref[...], preferred_element_type=f32)
                       + ts_b2_ref[...]).reshape(1, L, 384)

    hi = jnp.dot(x, wa_ref[...], preferred_element_type=f32)
    hj = jnp.dot(x, wb_ref[...], preferred_element_type=f32)
    wc = wc_ref[...]
    bf16 = jnp.bfloat16
    corr = dcol * wc[0:1, :] + scol * wc[2:3, :]
    ai_ref[...] = (hi + corr).astype(bf16)
    bj_ref[...] = (hj - corr + tp_b1_ref[...]).astype(bf16)
    d1_ref[...] = (dcol * wc[1:2, :]).astype(bf16)
    s1_ref[...] = (scol * wc[3:4, :]).astype(bf16)
    d_ref[...] = dcol
    s_ref[...] = scol
    dsc_ref[...] = jnp.concatenate([jnp.broadcast_to(dcol, (L, 128)),
                                    jnp.broadcast_to(scol, (L, 128))], axis=1)

    w2 = tp_w2_ref[...]
    w2g = w2 * tp_gc_ref[...]
    w2a_ref[...] = w2g.astype(bf16)
    u_ref[...] = jnp.sum(w2g, axis=0, keepdims=True)
    cw2_ref[...] = (jnp.dot(tp_be_ref[...], w2, preferred_element_type=f32)
                    + tp_b2_ref[...])


def _pair_kernel(ai_ref, d1_ref, s1_ref, bj_ref, dsc_ref,
                 w2a_ref, u_ref, cw2_ref, w3_ref, b3_ref, o_ref):
    f32 = jnp.float32
    bf16 = jnp.bfloat16
    ai = ai_ref[...]
    bj = bj_ref[...]
    dj = dsc_ref[:, 0:1].astype(bf16)
    sj = dsc_ref[:, 128:129].astype(bf16)

    h = (ai[:, None, :] + bj[None, :, :]
         + d1_ref[...][:, None, :] * dj[None, :, :]
         + s1_ref[...][:, None, :] * sj[None, :, :])
    rb = jax.nn.relu(h).reshape(TI * TJ, 512)
    t = jnp.dot(rb, w2a_ref[...], preferred_element_type=f32)
    r = rb.astype(f32)
    s1 = jnp.sum(r, axis=-1, keepdims=True)
    s2 = jnp.sum(r * r, axis=-1, keepdims=True)
    m = s1 * (1.0 / 512.0)
    v = s2 * (1.0 / 512.0) - m * m
    inv = jax.lax.rsqrt(v + EPS)
    h2 = jax.nn.relu(inv * (t - m * u_ref[...]) + cw2_ref[...])
    o = jnp.dot(h2, w3_ref[...], preferred_element_type=f32) + b3_ref[...]
    o_ref[...] = o.reshape(1, TI, TJ, 128)


@jax.jit
def kernel(features, sp_w, sp_b, sp_g, sp_be, dms_w1, dms_b1, dms_w2, dms_b2,
           sh_w1, sh_b1, sh_w2, sh_b2, ts_w1, ts_b1, ts_g, ts_be, ts_w2, ts_b2,
           tp_w1, tp_b1, tp_g, tp_be, tp_w2, tp_b2, tp_w3, tp_b3):
    f32 = jnp.float32
    x = features.reshape(L, D)

    row = lambda v: v.reshape(1, -1)
    wa = tp_w1[:640]
    wb = tp_w1[640:1280]
    wc = jnp.pad(tp_w1[1280:1284], ((0, 4), (0, 0)))

    vmem = lambda: pl.BlockSpec(memory_space=pltpu.VMEM)
    single, dcol, scol, ai, bj, d1, s1, dsc, w2a, u, cw2 = pl.pallas_call(
        _rows_kernel,
        out_shape=[
            jax.ShapeDtypeStruct((1, L, 384), f32),
            jax.ShapeDtypeStruct((L, 1), f32),
            jax.ShapeDtypeStruct((L, 1), f32),
            jax.ShapeDtypeStruct((L, 512), jnp.bfloat16),
            jax.ShapeDtypeStruct((L, 512), jnp.bfloat16),
            jax.ShapeDtypeStruct((L, 512), jnp.bfloat16),
            jax.ShapeDtypeStruct((L, 512), jnp.bfloat16),
            jax.ShapeDtypeStruct((L, 256), f32),
            jax.ShapeDtypeStruct((512, 256), jnp.bfloat16),
            jax.ShapeDtypeStruct((1, 256), f32),
            jax.ShapeDtypeStruct((1, 256), f32),
        ],
        in_specs=[vmem() for _ in range(27)],
        out_specs=[vmem() for _ in range(11)],
    )(x, sp_w, row(sp_b), row(sp_g), row(sp_be),
      dms_w1, row(dms_b1), dms_w2.reshape(1, 160), dms_b2.reshape(1, 1),
      sh_w1, row(sh_b1), sh_w2.reshape(1, 160), sh_b2.reshape(1, 1),
      ts_w1, row(ts_b1), row(ts_g), row(ts_be), ts_w2, row(ts_b2),
      wa, wb, wc, row(tp_b1),
      tp_g.reshape(512, 1), row(tp_be), tp_w2, row(tp_b2))

    grid = (L // TI, L // TJ)
    pair = pl.pallas_call(
        _pair_kernel,
        grid=grid,
        in_specs=[
            pl.BlockSpec((TI, 512), lambda i, j: (i, 0)),
            pl.BlockSpec((TI, 512), lambda i, j: (i, 0)),
            pl.BlockSpec((TI, 512), lambda i, j: (i, 0)),
            pl.BlockSpec((TJ, 512), lambda i, j: (j, 0)),
            pl.BlockSpec((TJ, 256), lambda i, j: (j, 0)),
            pl.BlockSpec((512, 256), lambda i, j: (0, 0)),
            pl.BlockSpec((1, 256), lambda i, j: (0, 0)),
            pl.BlockSpec((1, 256), lambda i, j: (0, 0)),
            pl.BlockSpec((256, 128), lambda i, j: (0, 0)),
            pl.BlockSpec((1, 128), lambda i, j: (0, 0)),
        ],
        out_specs=pl.BlockSpec((1, TI, TJ, 128), lambda i, j: (0, i, j, 0)),
        out_shape=jax.ShapeDtypeStruct((1, L, L, 128), f32),
        compiler_params=pltpu.CompilerParams(
            dimension_semantics=("parallel", "arbitrary"),
            vmem_limit_bytes=56 * 1024 * 1024,
            flags={"XLA_TPU_STORE_TO_LOAD_FORWARDING_WINDOW": 12288},
        ),
    )(ai, d1, s1, bj, dsc, w2a, u, cw2, tp_w3, row(tp_b3))

    return (single, pair, dcol.reshape(1, L), scol.reshape(1, L))
